# Initial kernel scaffold; baseline (speedup 1.0000x reference)
import jax
import jax.numpy as jnp
from jax.experimental import pallas as pl


def kernel(features, edge_index):
    raise NotImplementedError("write your pallas kernel here")


# speedup vs baseline: 4.7380x; 4.7380x over previous
import functools

import jax
import jax.numpy as jnp
from jax import lax
from jax.experimental import pallas as pl
from jax.experimental.pallas import tpu as pltpu
from jax.experimental.pallas import tpu_sc as plsc

N_NODES = 10000
N_EDGES = 320000
D_FEAT = 128

NC = 2
NS = 16
NW = NC * NS
EDGES_PER_TILE = N_EDGES // NW
CHUNK = 80
NUM_CHUNKS = EDGES_PER_TILE // CHUNK
N_PAD = 10240
ROWS_PER_TILE = N_PAD // NS

_mesh = plsc.VectorSubcoreMesh(core_axis_name="c", subcore_axis_name="s")


@functools.partial(
    pl.kernel,
    out_type=jax.ShapeDtypeStruct((NC * N_PAD, D_FEAT), jnp.float32),
    mesh=_mesh,
    scratch_types=[
        pltpu.VMEM((CHUNK,), jnp.int32),
        pltpu.VMEM((CHUNK,), jnp.int32),
        pltpu.VMEM((CHUNK, D_FEAT), jnp.float32),
        pltpu.VMEM_SHARED((N_PAD, D_FEAT), jnp.float32),
        pltpu.SemaphoreType.DMA,
    ],
)
def _sc_segment_sum(feat_hbm, src_hbm, dst_hbm, zero_hbm, part_hbm,
                    src_v, dst_v, rows_v, acc, sem):
    c = lax.axis_index("c")
    s = lax.axis_index("s")
    wid = c * NS + s

    r0 = s * ROWS_PER_TILE
    pltpu.sync_copy(zero_hbm.at[pl.ds(r0, ROWS_PER_TILE)],
                    acc.at[pl.ds(r0, ROWS_PER_TILE)])
    plsc.subcore_barrier()

    ebase = wid * EDGES_PER_TILE

    def step(i, carry):
        base = ebase + i * CHUNK
        pltpu.sync_copy(src_hbm.at[pl.ds(base, CHUNK)], src_v)
        pltpu.sync_copy(dst_hbm.at[pl.ds(base, CHUNK)], dst_v)
        pltpu.async_copy(feat_hbm.at[src_v], rows_v, sem).wait()
        pltpu.sync_copy(rows_v, acc.at[dst_v], add=True)
        return carry

    lax.fori_loop(0, NUM_CHUNKS, step, 0)
    plsc.subcore_barrier()

    pltpu.sync_copy(acc.at[pl.ds(r0, ROWS_PER_TILE)],
                    part_hbm.at[pl.ds(c * N_PAD + r0, ROWS_PER_TILE)])


def _combine_body(a_ref, b_ref, o_ref):
    o_ref[...] = a_ref[...] + b_ref[...]


_BLK = 80
_N_BLK = N_NODES // _BLK
_PAD_BLKS = N_PAD // _BLK


def _combine(partial):
    return pl.pallas_call(
        _combine_body,
        out_shape=jax.ShapeDtypeStruct((N_NODES, D_FEAT), jnp.float32),
        grid=(_N_BLK,),
        in_specs=[
            pl.BlockSpec((_BLK, D_FEAT), lambda i: (i, 0)),
            pl.BlockSpec((_BLK, D_FEAT), lambda i: (i + _PAD_BLKS, 0)),
        ],
        out_specs=pl.BlockSpec((_BLK, D_FEAT), lambda i: (i, 0)),
    )(partial, partial)


def kernel(features, edge_index):
    src = edge_index[0].astype(jnp.int32)
    dst = edge_index[1].astype(jnp.int32)
    zeros = jnp.zeros((N_PAD, D_FEAT), jnp.float32)
    partial = _sc_segment_sum(features, src, dst, zeros)
    return _combine(partial)


# speedup vs baseline: 9.2521x; 1.9528x over previous
import functools

import jax
import jax.numpy as jnp
from jax import lax
from jax.experimental import pallas as pl
from jax.experimental.pallas import tpu as pltpu
from jax.experimental.pallas import tpu_sc as plsc

N_NODES = 10000
N_EDGES = 320000
D_FEAT = 128

NC = 2
NS = 16
NW = NC * NS
CHUNK = 125
N_CHUNKS = N_EDGES // CHUNK
CPT = N_CHUNKS // NW
N_PAD = 10240
ROWS_PER_TILE = N_PAD // NS

_mesh = plsc.VectorSubcoreMesh(core_axis_name="c", subcore_axis_name="s")


@functools.partial(
    pl.kernel,
    out_type=jax.ShapeDtypeStruct((NC * N_PAD, D_FEAT), jnp.float32),
    mesh=_mesh,
    scratch_types=[
        pltpu.VMEM((CPT, CHUNK), jnp.int32),
        pltpu.VMEM((CHUNK,), jnp.int32),
        pltpu.VMEM((CHUNK,), jnp.int32),
        pltpu.VMEM((CHUNK, D_FEAT), jnp.float32),
        pltpu.VMEM((CHUNK, D_FEAT), jnp.float32),
        pltpu.VMEM_SHARED((N_PAD, D_FEAT), jnp.float32),
        pltpu.SemaphoreType.DMA,
        pltpu.SemaphoreType.DMA,
        pltpu.SemaphoreType.DMA,
        pltpu.SemaphoreType.DMA,
    ],
)
def _sc_segment_sum(feat_hbm, src_hbm, dst_hbm, zero_hbm, part_hbm,
                    src_all, d0, d1, rows0, rows1, acc,
                    sem0, sem1, semd0, semd1):
    c = lax.axis_index("c")
    s = lax.axis_index("s")
    wid = c * NS + s
    cbase = wid * CPT

    pltpu.sync_copy(src_hbm.at[pl.ds(cbase, CPT)], src_all)

    r0 = s * ROWS_PER_TILE
    pltpu.sync_copy(zero_hbm.at[pl.ds(r0, ROWS_PER_TILE)],
                    acc.at[pl.ds(r0, ROWS_PER_TILE)])
    plsc.subcore_barrier()

    def gather_start(chunk, rows, sem):
        pltpu.async_copy(feat_hbm.at[src_all.at[chunk]], rows, sem)

    def gather_wait(chunk, rows, sem):
        pltpu.make_async_copy(feat_hbm.at[src_all.at[chunk]], rows, sem).wait()

    def dst_start(chunk, d, sem):
        pltpu.async_copy(dst_hbm.at[cbase + chunk], d, sem)

    def dst_wait(chunk, d, sem):
        pltpu.make_async_copy(dst_hbm.at[cbase + chunk], d, sem).wait()

    def scatter(d, rows):
        pltpu.sync_copy(rows, acc.at[d], add=True)

    gather_start(0, rows0, sem0)
    dst_start(0, d0, semd0)

    def step(k, carry):
        e = 2 * k
        gather_start(e + 1, rows1, sem1)
        dst_start(e + 1, d1, semd1)
        gather_wait(e, rows0, sem0)
        dst_wait(e, d0, semd0)
        scatter(d0, rows0)
        gather_start(e + 2, rows0, sem0)
        dst_start(e + 2, d0, semd0)
        gather_wait(e + 1, rows1, sem1)
        dst_wait(e + 1, d1, semd1)
        scatter(d1, rows1)
        return carry

    lax.fori_loop(0, CPT // 2 - 1, step, 0)

    gather_start(CPT - 1, rows1, sem1)
    dst_start(CPT - 1, d1, semd1)
    gather_wait(CPT - 2, rows0, sem0)
    dst_wait(CPT - 2, d0, semd0)
    scatter(d0, rows0)
    gather_wait(CPT - 1, rows1, sem1)
    dst_wait(CPT - 1, d1, semd1)
    scatter(d1, rows1)
    plsc.subcore_barrier()

    pltpu.sync_copy(acc.at[pl.ds(r0, ROWS_PER_TILE)],
                    part_hbm.at[pl.ds(c * N_PAD + r0, ROWS_PER_TILE)])


def _combine_body(a_ref, b_ref, o_ref):
    o_ref[...] = a_ref[...] + b_ref[...]


_BLK = 80
_N_BLK = N_NODES // _BLK
_PAD_BLKS = N_PAD // _BLK


def _combine(partial):
    return pl.pallas_call(
        _combine_body,
        out_shape=jax.ShapeDtypeStruct((N_NODES, D_FEAT), jnp.float32),
        grid=(_N_BLK,),
        in_specs=[
            pl.BlockSpec((_BLK, D_FEAT), lambda i: (i, 0)),
            pl.BlockSpec((_BLK, D_FEAT), lambda i: (i + _PAD_BLKS, 0)),
        ],
        out_specs=pl.BlockSpec((_BLK, D_FEAT), lambda i: (i, 0)),
    )(partial, partial)


def kernel(features, edge_index):
    src = edge_index[0].astype(jnp.int32).reshape(N_CHUNKS, CHUNK)
    dst = edge_index[1].astype(jnp.int32).reshape(N_CHUNKS, CHUNK)
    zeros = jnp.zeros((N_PAD, D_FEAT), jnp.float32)
    partial = _sc_segment_sum(features, src, dst, zeros)
    return _combine(partial)


# speedup vs baseline: 9.6572x; 1.0438x over previous
import functools

import jax
import jax.numpy as jnp
from jax import lax
from jax.experimental import pallas as pl
from jax.experimental.pallas import tpu as pltpu
from jax.experimental.pallas import tpu_sc as plsc

N_NODES = 10000
N_EDGES = 320000
D_FEAT = 128

NC = 2
NS = 16
NW = NC * NS
EPT = N_EDGES // NW
CHUNK = 80
CPT = EPT // CHUNK
N_PAD = 10240
ROWS_PER_TILE = N_PAD // NS
NBUF = 3

_mesh = plsc.VectorSubcoreMesh(core_axis_name="c", subcore_axis_name="s")


@functools.partial(
    pl.kernel,
    out_type=jax.ShapeDtypeStruct((NC * N_PAD, D_FEAT), jnp.float32),
    mesh=_mesh,
    scratch_types=[
        pltpu.VMEM((EPT,), jnp.int32),
        [pltpu.VMEM((CHUNK,), jnp.int32) for _ in range(NBUF)],
        [pltpu.VMEM((CHUNK, D_FEAT), jnp.float32) for _ in range(NBUF)],
        pltpu.VMEM_SHARED((N_PAD, D_FEAT), jnp.float32),
        [pltpu.SemaphoreType.DMA for _ in range(3 * NBUF)],
    ],
)
def _sc_segment_sum(feat_hbm, src_hbm, dst_hbm, zero_hbm, part_hbm,
                    src_all, dbufs, rbufs, acc, sems):
    c = lax.axis_index("c")
    s = lax.axis_index("s")
    wid = c * NS + s
    cbase = wid * CPT
    gsems, dsems, ssems = sems[0:NBUF], sems[NBUF:2 * NBUF], sems[2 * NBUF:]

    pltpu.sync_copy(src_hbm.at[pl.ds(wid * EPT, EPT)], src_all)

    r0 = s * ROWS_PER_TILE
    pltpu.sync_copy(zero_hbm.at[pl.ds(r0, ROWS_PER_TILE)],
                    acc.at[pl.ds(r0, ROWS_PER_TILE)])
    plsc.subcore_barrier()

    def gather_start(e, m):
        pltpu.async_copy(feat_hbm.at[src_all.at[pl.ds(e * CHUNK, CHUNK)]],
                         rbufs[m], gsems[m])

    def gather_wait(e, m):
        pltpu.make_async_copy(feat_hbm.at[src_all.at[pl.ds(e * CHUNK, CHUNK)]],
                              rbufs[m], gsems[m]).wait()

    def dst_start(e, m):
        pltpu.async_copy(dst_hbm.at[cbase + e], dbufs[m], dsems[m])

    def dst_wait(e, m):
        pltpu.make_async_copy(dst_hbm.at[cbase + e], dbufs[m], dsems[m]).wait()

    def scatter_start(m):
        pltpu.async_copy(rbufs[m], acc.at[dbufs[m]], ssems[m], add=True)

    def scatter_wait(m):
        pltpu.make_async_copy(rbufs[m], acc.at[dbufs[m]], ssems[m]).wait()

    def slot(e, m, tail_wait=True, prefetch=True):
        gather_wait(e, m)
        dst_wait(e, m)
        scatter_start(m)
        if tail_wait:
            scatter_wait((m + 2) % NBUF)
        if prefetch:
            gather_start(e + 2, (m + 2) % NBUF)
            dst_start(e + 2, (m + 2) % NBUF)

    gather_start(0, 0)
    dst_start(0, 0)
    gather_start(1, 1)
    dst_start(1, 1)

    slot(0, 0, tail_wait=False)
    slot(1, 1)
    slot(2, 2)

    def step(k, carry):
        e = 3 * k + 3
        slot(e, 0)
        slot(e + 1, 1)
        slot(e + 2, 2)
        return carry

    lax.fori_loop(0, (CPT - 5) // 3, step, 0)

    slot(CPT - 2, (CPT - 2) % NBUF, prefetch=False)
    slot(CPT - 1, (CPT - 1) % NBUF, prefetch=False)
    scatter_wait((CPT - 1) % NBUF)
    plsc.subcore_barrier()

    pltpu.sync_copy(acc.at[pl.ds(r0, ROWS_PER_TILE)],
                    part_hbm.at[pl.ds(c * N_PAD + r0, ROWS_PER_TILE)])


def _combine_body(a_ref, b_ref, o_ref):
    o_ref[...] = a_ref[...] + b_ref[...]


_BLK = 80
_N_BLK = N_NODES // _BLK
_PAD_BLKS = N_PAD // _BLK


def _combine(partial):
    return pl.pallas_call(
        _combine_body,
        out_shape=jax.ShapeDtypeStruct((N_NODES, D_FEAT), jnp.float32),
        grid=(_N_BLK,),
        in_specs=[
            pl.BlockSpec((_BLK, D_FEAT), lambda i: (i, 0)),
            pl.BlockSpec((_BLK, D_FEAT), lambda i: (i + _PAD_BLKS, 0)),
        ],
        out_specs=pl.BlockSpec((_BLK, D_FEAT), lambda i: (i, 0)),
    )(partial, partial)


def kernel(features, edge_index):
    src = edge_index[0].astype(jnp.int32)
    dst = edge_index[1].astype(jnp.int32).reshape(N_EDGES // CHUNK, CHUNK)
    zeros = jnp.zeros((N_PAD, D_FEAT), jnp.float32)
    partial = _sc_segment_sum(features, src, dst, zeros)
    return _combine(partial)
